# baseline (speedup 1.0000x reference)
import jax
import jax.numpy as jnp
from jax.experimental import pallas as pl
from jax.experimental.pallas import tpu as pltpu

N_WAY = 200
N_SUP = 1000
N_Q = 4096
N_SAMP = 8
FEAT = 2048
BQ = 256
BS = 200
CPAD = 256


def _support_mean_kernel(sup_ref, sm_ref):
    sm_ref[...] = jnp.mean(sup_ref[...], axis=1)


def _dist_kernel(q_ref, sm_ref, lab_ref, out_ref):
    qm = jnp.mean(q_ref[...], axis=1)
    sm = sm_ref[...]
    q2 = jnp.sum(qm * qm, axis=1, keepdims=True)
    s2 = jnp.sum(sm * sm, axis=1, keepdims=True)

    lhs = jnp.concatenate(
        [(-2.0 * qm).astype(jnp.bfloat16),
         jnp.ones((qm.shape[0], 1), jnp.bfloat16)], axis=1)
    rhs = jnp.concatenate(
        [sm.astype(jnp.bfloat16), s2.astype(jnp.bfloat16)], axis=1)
    dots = jax.lax.dot_general(
        lhs, rhs, (((1,), (1,)), ((), ())),
        preferred_element_type=jnp.float32)
    dist = jnp.sqrt(jnp.maximum(q2 + dots, 1e-12))

    lab = lab_ref[...]
    cls = jax.lax.broadcasted_iota(jnp.int32, (N_SUP, CPAD), 1)
    onehot = (lab == cls).astype(jnp.bfloat16)
    sums = jax.lax.dot_general(
        dist.astype(jnp.bfloat16), onehot, (((1,), (0,)), ((), ())),
        preferred_element_type=jnp.float32)
    counts = jnp.sum(onehot.astype(jnp.float32), axis=0, keepdims=True)
    scale = jnp.where(counts > 0, -1.0 / counts, 0.0)
    out_ref[...] = (sums * scale)[:, :N_WAY]


def kernel(support_set, support_labels, queries):
    sm = pl.pallas_call(
        _support_mean_kernel,
        grid=(N_SUP // BS,),
        in_specs=[pl.BlockSpec((BS, N_SAMP, FEAT), lambda i: (i, 0, 0))],
        out_specs=pl.BlockSpec((BS, FEAT), lambda i: (i, 0)),
        out_shape=jax.ShapeDtypeStruct((N_SUP, FEAT), jnp.float32),
        compiler_params=pltpu.CompilerParams(
            dimension_semantics=("arbitrary",)),
    )(support_set)

    lab_col = support_labels.astype(jnp.int32).reshape(N_SUP, 1)
    out = pl.pallas_call(
        _dist_kernel,
        grid=(N_Q // BQ,),
        in_specs=[
            pl.BlockSpec((BQ, N_SAMP, FEAT), lambda i: (i, 0, 0)),
            pl.BlockSpec((N_SUP, FEAT), lambda i: (0, 0)),
            pl.BlockSpec((N_SUP, 1), lambda i: (0, 0)),
        ],
        out_specs=pl.BlockSpec((BQ, N_WAY), lambda i: (i, 0)),
        out_shape=jax.ShapeDtypeStruct((N_Q, N_WAY), jnp.float32),
        compiler_params=pltpu.CompilerParams(
            dimension_semantics=("arbitrary",)),
    )(queries, sm, lab_col)
    return out


# speedup vs baseline: 1.6108x; 1.6108x over previous
import jax
import jax.numpy as jnp
from jax.experimental import pallas as pl
from jax.experimental.pallas import tpu as pltpu

N_WAY = 200
N_SUP = 1000
N_Q = 4096
N_SAMP = 8
FEAT = 2048
BQ = 256
BS = 200
CPAD = 256


def _sel_matrix(rows):
    r = jax.lax.broadcasted_iota(jnp.int32, (rows, rows * N_SAMP), 1)
    c = jax.lax.broadcasted_iota(jnp.int32, (rows, rows * N_SAMP), 0)
    return jnp.where(r // N_SAMP == c, -0.25, 0.0).astype(jnp.bfloat16)


def _support_kernel(supf_ref, rhs_ref):
    supf = supf_ref[...].astype(jnp.bfloat16)
    r = jax.lax.broadcasted_iota(jnp.int32, (BS, BS * N_SAMP), 1)
    c = jax.lax.broadcasted_iota(jnp.int32, (BS, BS * N_SAMP), 0)
    sel = jnp.where(r // N_SAMP == c, 0.125, 0.0).astype(jnp.bfloat16)
    smf = jax.lax.dot_general(
        sel, supf, (((1,), (0,)), ((), ())),
        preferred_element_type=jnp.float32)
    s2 = jnp.sum(smf * smf, axis=1, keepdims=True)
    rhs_ref[...] = jnp.concatenate(
        [smf.astype(jnp.bfloat16), s2.astype(jnp.bfloat16)], axis=1)


def _dist_kernel(qf_ref, rhs_ref, lab_ref, out_ref, sel_scr, oh_scr, scale_scr):
    @pl.when(pl.program_id(0) == 0)
    def _init():
        sel_scr[...] = _sel_matrix(BQ)
        lab = lab_ref[...]
        cls = jax.lax.broadcasted_iota(jnp.int32, (N_SUP, CPAD), 1)
        oh = lab == cls
        oh_scr[...] = oh.astype(jnp.bfloat16)
        counts = jnp.sum(oh.astype(jnp.float32), axis=0, keepdims=True)
        scale_scr[...] = jnp.where(counts > 0, -1.0 / counts, 0.0)

    qf = qf_ref[...].astype(jnp.bfloat16)
    qm2 = jax.lax.dot_general(
        sel_scr[...], qf, (((1,), (0,)), ((), ())),
        preferred_element_type=jnp.float32)
    q2 = 0.25 * jnp.sum(qm2 * qm2, axis=1, keepdims=True)
    lhs = jnp.concatenate(
        [qm2.astype(jnp.bfloat16),
         jnp.ones((BQ, 1), jnp.bfloat16)], axis=1)
    dots = jax.lax.dot_general(
        lhs, rhs_ref[...], (((1,), (1,)), ((), ())),
        preferred_element_type=jnp.float32)
    dist = jnp.sqrt(jnp.maximum(q2 + dots, 1e-12)).astype(jnp.bfloat16)
    sums = jax.lax.dot_general(
        dist, oh_scr[...], (((1,), (0,)), ((), ())),
        preferred_element_type=jnp.float32)
    out_ref[...] = (sums * scale_scr[...])[:, :N_WAY]


def kernel(support_set, support_labels, queries):
    supf = support_set.reshape(N_SUP * N_SAMP, FEAT)
    rhs = pl.pallas_call(
        _support_kernel,
        grid=(N_SUP // BS,),
        in_specs=[pl.BlockSpec((BS * N_SAMP, FEAT), lambda i: (i, 0))],
        out_specs=pl.BlockSpec((BS, FEAT + 1), lambda i: (i, 0)),
        out_shape=jax.ShapeDtypeStruct((N_SUP, FEAT + 1), jnp.bfloat16),
        compiler_params=pltpu.CompilerParams(
            dimension_semantics=("arbitrary",)),
    )(supf)

    qf = queries.reshape(N_Q * N_SAMP, FEAT)
    lab_col = support_labels.astype(jnp.int32).reshape(N_SUP, 1)
    out = pl.pallas_call(
        _dist_kernel,
        grid=(N_Q // BQ,),
        in_specs=[
            pl.BlockSpec((BQ * N_SAMP, FEAT), lambda i: (i, 0)),
            pl.BlockSpec((N_SUP, FEAT + 1), lambda i: (0, 0)),
            pl.BlockSpec((N_SUP, 1), lambda i: (0, 0)),
        ],
        out_specs=pl.BlockSpec((BQ, N_WAY), lambda i: (i, 0)),
        out_shape=jax.ShapeDtypeStruct((N_Q, N_WAY), jnp.float32),
        scratch_shapes=[
            pltpu.VMEM((BQ, BQ * N_SAMP), jnp.bfloat16),
            pltpu.VMEM((N_SUP, CPAD), jnp.bfloat16),
            pltpu.VMEM((1, CPAD), jnp.float32),
        ],
        compiler_params=pltpu.CompilerParams(
            dimension_semantics=("arbitrary",)),
    )(qf, rhs, lab_col)
    return out
